# baseline (speedup 1.0000x reference)
import functools

import jax
import jax.numpy as jnp
from jax import lax
from jax.experimental import pallas as pl
from jax.experimental.pallas import tpu as pltpu
from jax.experimental.pallas import tpu_sc as plsc

_K = 1000
_OFF0 = 7 * _K + 1
_OFF1 = 23 * _K + 1
_D = 1024
_L = 2
_B = 4


def _vq_kernel(f_ref, cb_ref, tok_ref, pr_ref, idx_ref, csq_ref):
    @pl.when(pl.program_id(0) == 0)
    def _init_csq():
        for l in range(_L):
            cb = cb_ref[l * _K:(l + 1) * _K, :]
            csq_ref[l, :] = jnp.sum(cb * cb, axis=1)

    tt = f_ref.shape[1]
    rows = _B * tt
    toks = []
    for l in range(_L):
        fl = f_ref[:, :, l, :].reshape(rows, _D)
        cb = cb_ref[l * _K:(l + 1) * _K, :]
        dots = jax.lax.dot_general(
            fl, cb, (((1,), (1,)), ((), ())),
            preferred_element_type=jnp.float32)
        dist = csq_ref[l, :][None, :] - 2.0 * dots
        tok = jnp.argmin(dist, axis=1).astype(jnp.int32)
        toks.append(tok)
    tok2 = jnp.stack(toks, axis=1)
    tok_ref[0, :, :] = tok2
    colt = jax.lax.broadcasted_iota(jnp.int32, tok2.shape, 1)
    pr_ref[0, :, :] = tok2 + _OFF0 + colt * (_OFF1 - _OFF0)
    idx_ref[0, :, :] = tok2 + colt * _K


_NROWS = _B * 1500 * _L
_WORKERS = 30
_PER_W = _NROWS // _WORKERS
_CHUNK = 16
_NCHUNK = _PER_W // _CHUNK


def _sc_gather(table, idx):
    mesh = plsc.VectorSubcoreMesh(core_axis_name="c", subcore_axis_name="s")

    @functools.partial(
        pl.kernel, mesh=mesh,
        out_type=jax.ShapeDtypeStruct((_NROWS, _D), jnp.float32),
        scratch_types=[
            pltpu.VMEM((_CHUNK,), jnp.int32),
            pltpu.VMEM((_CHUNK, _D), jnp.float32),
            pltpu.SemaphoreType.DMA,
        ],
    )
    def k(table_hbm, idx_hbm, out_hbm, idx_v, rows_v, sem):
        info = plsc.get_sparse_core_info()
        wid = lax.axis_index("s") * info.num_cores + lax.axis_index("c")

        @pl.when(wid < _WORKERS)
        def _work():
            base = wid * _PER_W

            @pl.loop(0, _NCHUNK)
            def _chunk(j):
                off = base + j * _CHUNK
                pltpu.sync_copy(idx_hbm.at[pl.ds(off, _CHUNK)], idx_v)
                pltpu.async_copy(table_hbm.at[idx_v], rows_v, sem).wait()
                pltpu.sync_copy(rows_v, out_hbm.at[pl.ds(off, _CHUNK)])

    return k(table, idx)


@jax.jit
def kernel(feats, codebooks):
    B, T, L, D = feats.shape
    K = codebooks.shape[1]
    cb2 = codebooks.reshape(L * K, D)
    tt = 250
    nst = T // tt
    tok2, pr2, idx3 = pl.pallas_call(
        _vq_kernel,
        grid=(nst,),
        in_specs=[
            pl.BlockSpec((B, tt, L, D), lambda i: (0, i, 0, 0)),
            pl.BlockSpec((L * K, D), lambda i: (0, 0)),
        ],
        out_specs=[
            pl.BlockSpec((1, B * tt, L), lambda i: (i, 0, 0)),
            pl.BlockSpec((1, B * tt, L), lambda i: (i, 0, 0)),
            pl.BlockSpec((1, B * tt, L), lambda i: (i, 0, 0)),
        ],
        out_shape=[
            jax.ShapeDtypeStruct((nst, B * tt, L), jnp.int32),
            jax.ShapeDtypeStruct((nst, B * tt, L), jnp.int32),
            jax.ShapeDtypeStruct((nst, B * tt, L), jnp.int32),
        ],
        scratch_shapes=[pltpu.VMEM((L, _K), jnp.float32)],
    )(feats, cb2)
    tokens = (tok2.reshape(nst, B, tt, L)
              .transpose(1, 0, 2, 3).reshape(B, T, L))
    pr_tokens = (pr2.reshape(nst, B, tt, L)
                 .transpose(1, 0, 2, 3).reshape(B, T, L))
    idx_flat = (idx3.reshape(nst, B, tt, L)
                .transpose(1, 0, 2, 3).reshape(B * T * L))
    embs = _sc_gather(cb2, idx_flat).reshape(B, T, L, D)
    return tokens, embs, pr_tokens


# speedup vs baseline: 2.2710x; 2.2710x over previous
import jax
import jax.numpy as jnp
from jax.experimental import pallas as pl
from jax.experimental.pallas import tpu as pltpu

_K = 1000
_OFF0 = 7 * _K + 1
_OFF1 = 23 * _K + 1
_D = 1024
_L = 2
_B = 4


def _vq_kernel(f_ref, cb_ref, tok_ref, emb_ref, pr_ref, csq_ref):
    @pl.when(pl.program_id(0) == 0)
    def _init_csq():
        for l in range(_L):
            cb = cb_ref[l * _K:(l + 1) * _K, :]
            csq_ref[l, :] = jnp.sum(cb * cb, axis=1)

    tt = f_ref.shape[1]
    rows = _B * tt
    toks = []
    for l in range(_L):
        fl = f_ref[:, :, l, :].reshape(rows, _D)
        cb = cb_ref[l * _K:(l + 1) * _K, :]
        dots = jax.lax.dot_general(
            fl, cb, (((1,), (1,)), ((), ())),
            preferred_element_type=jnp.float32)
        dist = csq_ref[l, :][None, :] - 2.0 * dots
        tok = jnp.argmin(dist, axis=1).astype(jnp.int32)
        col = jax.lax.broadcasted_iota(jnp.int32, dots.shape, 1)
        one_hot = (col == tok[:, None]).astype(jnp.float32)
        emb = jax.lax.dot_general(
            one_hot, cb, (((1,), (0,)), ((), ())),
            preferred_element_type=jnp.float32)
        emb_ref[:, :, l, :] = emb.reshape(_B, tt, _D)
        toks.append(tok)
    tok2 = jnp.stack(toks, axis=1)
    tok_ref[0, :, :] = tok2
    colt = jax.lax.broadcasted_iota(jnp.int32, tok2.shape, 1)
    pr_ref[0, :, :] = tok2 + _OFF0 + colt * (_OFF1 - _OFF0)


@jax.jit
def kernel(feats, codebooks):
    B, T, L, D = feats.shape
    K = codebooks.shape[1]
    rows = B * T
    cb2 = codebooks.reshape(L * K, D)
    tt = 250
    grid = (T // tt,)
    tok2, embs, pr2 = pl.pallas_call(
        _vq_kernel,
        grid=grid,
        in_specs=[
            pl.BlockSpec((B, tt, L, D), lambda i: (0, i, 0, 0)),
            pl.BlockSpec((L * K, D), lambda i: (0, 0)),
        ],
        out_specs=[
            pl.BlockSpec((1, B * tt, L), lambda i: (i, 0, 0)),
            pl.BlockSpec((B, tt, L, D), lambda i: (0, i, 0, 0)),
            pl.BlockSpec((1, B * tt, L), lambda i: (i, 0, 0)),
        ],
        out_shape=[
            jax.ShapeDtypeStruct((T // tt, B * tt, L), jnp.int32),
            jax.ShapeDtypeStruct((B, T, L, D), jnp.float32),
            jax.ShapeDtypeStruct((T // tt, B * tt, L), jnp.int32),
        ],
        scratch_shapes=[pltpu.VMEM((L, _K), jnp.float32)],
    )(feats, cb2)
    tokens = (tok2.reshape(T // tt, B, tt, L)
              .transpose(1, 0, 2, 3).reshape(B, T, L))
    pr_tokens = (pr2.reshape(T // tt, B, tt, L)
                 .transpose(1, 0, 2, 3).reshape(B, T, L))
    return tokens, embs, pr_tokens
